# baseline (speedup 1.0000x reference)
import functools

import jax
import jax.numpy as jnp
from jax import lax
from jax.experimental import pallas as pl
from jax.experimental.pallas import tpu as pltpu
from jax.experimental.pallas import tpu_sc as plsc

EMBED_DIM = 64
SZ0 = 104
SZ1 = 96
NBUF = 4
BUFS = 2 * NBUF
NUM_CORES = 2
NUM_SUBCORES = 16
NUM_WORKERS = NUM_CORES * NUM_SUBCORES


def _sz(i):
    return SZ0 if i % 2 == 0 else SZ1


def _off(i):
    return 0 if i % 2 == 0 else SZ0


@functools.lru_cache(maxsize=None)
def _build_gather(batch: int, hist: int):
    rows_per_worker = batch // NUM_WORKERS
    nchunk = rows_per_worker * 2
    assert hist == SZ0 + SZ1 and nchunk % BUFS == 0
    mesh = plsc.VectorSubcoreMesh(core_axis_name="c", subcore_axis_name="s")

    @functools.partial(
        pl.kernel,
        mesh=mesh,
        out_type=jax.ShapeDtypeStruct((batch, hist, EMBED_DIM), jnp.float32),
        scratch_types=[
            pltpu.VMEM((rows_per_worker, hist), jnp.int32),
            pltpu.VMEM((BUFS, SZ0, EMBED_DIM), jnp.float32),
            pltpu.SemaphoreType.DMA((BUFS,)),
            pltpu.SemaphoreType.DMA((BUFS,)),
        ],
        compiler_params=pltpu.CompilerParams(use_tc_tiling_on_sc=False),
    )
    def gather_kernel(idx_hbm, table_hbm, out_hbm, idx_v, rows_v, gsem, wsem):
        wid = lax.axis_index("c") * NUM_SUBCORES + lax.axis_index("s")
        bbase = wid * rows_per_worker

        pltpu.sync_copy(idx_hbm.at[pl.ds(bbase, rows_per_worker)], idx_v)

        def start_gather(r, i, b):
            pltpu.async_copy(
                table_hbm.at[idx_v.at[r, pl.ds(_off(i), _sz(i))]],
                rows_v.at[b, pl.ds(0, _sz(i))],
                gsem.at[b],
            )

        for b in range(NBUF):
            start_gather(b // 2, b, b)

        def body(jj, carry):
            for i in range(BUFS):
                c = jj * BUFS + i
                r = jj * (BUFS // 2) + i // 2
                pltpu.make_async_copy(
                    table_hbm.at[pl.ds(0, _sz(i))],
                    rows_v.at[i, pl.ds(0, _sz(i))],
                    gsem.at[i],
                ).wait()
                pltpu.async_copy(
                    rows_v.at[i, pl.ds(0, _sz(i))],
                    out_hbm.at[bbase + r, pl.ds(_off(i), _sz(i))],
                    wsem.at[i],
                )
                bn = (i + NBUF) % BUFS
                nxt = c + NBUF

                @pl.when(c >= NBUF)
                def _drain():
                    pltpu.make_async_copy(
                        rows_v.at[bn, pl.ds(0, _sz(i))],
                        out_hbm.at[0, pl.ds(0, _sz(i))],
                        wsem.at[bn],
                    ).wait()

                @pl.when(nxt < nchunk)
                def _next():
                    start_gather(jj * (BUFS // 2) + (i + NBUF) // 2, i, bn)

            return carry

        lax.fori_loop(0, nchunk // BUFS, body, 0)

        for k in range(NBUF):
            b = (nchunk - NBUF + k) % BUFS
            pltpu.make_async_copy(
                rows_v.at[b, pl.ds(0, _sz(b))],
                out_hbm.at[0, pl.ds(0, _sz(b))],
                wsem.at[b],
            ).wait()

    return gather_kernel


def kernel(token_ids, embeddings):
    b, h = token_ids.shape
    return _build_gather(b, h)(token_ids.astype(jnp.int32), embeddings)
